# baseline (speedup 1.0000x reference)
import jax
import jax.numpy as jnp
from jax.experimental import pallas as pl
from jax.experimental.pallas import tpu as pltpu


def _mlp_kernel(w1_ref, b1_ref, w2_ref, b2_ref, x_ref, o_ref):
    x = x_ref[...]
    h = jnp.dot(w1_ref[...], x, preferred_element_type=jnp.float32)
    h = jnp.maximum(h + b1_ref[...].T, 0.0)
    o = jnp.dot(w2_ref[...], h, preferred_element_type=jnp.float32)
    o_ref[...] = o + b2_ref[...].T


def kernel(x, w1, b1, w2, b2):
    B, F = x.shape
    C = w2.shape[0]
    xt = x.T
    b1c = b1.reshape(1, F)
    b2c = b2.reshape(1, C)

    tb = 65536
    while B % tb:
        tb //= 2
    grid = (B // tb,)

    out_t = pl.pallas_call(
        _mlp_kernel,
        out_shape=jax.ShapeDtypeStruct((C, B), jnp.float32),
        grid=grid,
        in_specs=[
            pl.BlockSpec((F, F), lambda i: (0, 0)),
            pl.BlockSpec((1, F), lambda i: (0, 0)),
            pl.BlockSpec((C, F), lambda i: (0, 0)),
            pl.BlockSpec((1, C), lambda i: (0, 0)),
            pl.BlockSpec((F, tb), lambda i: (0, i)),
        ],
        out_specs=pl.BlockSpec((C, tb), lambda i: (0, i)),
        compiler_params=pltpu.CompilerParams(
            dimension_semantics=("parallel",)),
    )(w1, b1c, w2, b2c, xt)
    return out_t.T


# speedup vs baseline: 1.0779x; 1.0779x over previous
import jax
import jax.numpy as jnp
from jax.experimental import pallas as pl
from jax.experimental.pallas import tpu as pltpu


def _mlp_kernel(w1_ref, b1_ref, w2_ref, b2_ref, x_ref, o_ref):
    x = x_ref[...]
    h = jnp.dot(w1_ref[...], x, preferred_element_type=jnp.float32)
    h = jnp.maximum(h + b1_ref[...].T, 0.0)
    o = jnp.dot(w2_ref[...], h, preferred_element_type=jnp.float32)
    o_ref[...] = o + b2_ref[...].T


def kernel(x, w1, b1, w2, b2):
    B, F = x.shape
    C = w2.shape[0]
    xt = x.T
    b1c = b1.reshape(1, F)
    b2c = b2.reshape(1, C)

    tb = 131072
    while B % tb:
        tb //= 2
    grid = (B // tb,)

    out_t = pl.pallas_call(
        _mlp_kernel,
        out_shape=jax.ShapeDtypeStruct((C, B), jnp.float32),
        grid=grid,
        in_specs=[
            pl.BlockSpec((F, F), lambda i: (0, 0)),
            pl.BlockSpec((1, F), lambda i: (0, 0)),
            pl.BlockSpec((C, F), lambda i: (0, 0)),
            pl.BlockSpec((1, C), lambda i: (0, 0)),
            pl.BlockSpec((F, tb), lambda i: (0, i)),
        ],
        out_specs=pl.BlockSpec((C, tb), lambda i: (0, i)),
        compiler_params=pltpu.CompilerParams(
            dimension_semantics=("parallel",)),
    )(w1, b1c, w2, b2c, xt)
    return out_t.T
